# baseline (speedup 1.0000x reference)
import jax
import jax.numpy as jnp
from jax import lax
from jax.experimental import pallas as pl
from jax.experimental.pallas import tpu as pltpu
from jax.experimental.pallas import tpu_sc as plsc

N = 10000
D = 128
H = 128
C = 40

NC = 2
NS = 16
NW = NC * NS
BATCH = 128
N_PAD = 10240
DUMP = N
RPT = N_PAD // NS
LANE = 16

_PREC = lax.Precision.HIGHEST


def _proj_body(x_ref, wl_ref, wr_ref, b_ref, p_ref, s_ref):
    xb = x_ref[...]
    p_ref[...] = jnp.dot(xb, wl_ref[...], precision=_PREC)
    s_ref[...] = jnp.dot(xb, wr_ref[...], precision=_PREC) + b_ref[...]


def _mid_body(acc_ref, wl_ref, wr_ref, b_ref, p_ref, s_ref):
    h = jnp.maximum(acc_ref[...], 0.0)
    z = jnp.zeros((h.shape[0], 64 - C), jnp.float32)
    p = jnp.concatenate([jnp.dot(h, wl_ref[...], precision=_PREC), z], axis=1)
    s = jnp.concatenate(
        [0.5 * (jnp.dot(h, wr_ref[...], precision=_PREC) + b_ref[...]), z], axis=1)
    p_ref[...] = jnp.concatenate([p, p], axis=1)
    s_ref[...] = jnp.concatenate([s, s], axis=1)


def _final_body(acc_ref, o_ref):
    o_ref[...] = acc_ref[:, :C] + acc_ref[:, 64:64 + C]


NBUF = 2


def _edge_loop(p_hbm, src_v, dst_v, acc_sh, bufs, gsems, T):
    def step(i, carry):
        j0 = 2 * i
        j1 = j0 + 1
        d0 = pltpu.async_copy(p_hbm.at[src_v.at[j0]], bufs[0], gsems[0])
        d1 = pltpu.async_copy(p_hbm.at[src_v.at[j1]], bufs[1], gsems[1])
        d0.wait()
        pltpu.sync_copy(bufs[0], acc_sh.at[dst_v.at[j0]], add=True)
        d1.wait()
        pltpu.sync_copy(bufs[1], acc_sh.at[dst_v.at[j1]], add=True)
        return carry

    lax.fori_loop(0, T // 2, step, 0)


def _transform_idx(src_v, c, n_rows):
    def step(t, carry):
        for k in range(BATCH // LANE):
            v = src_v[t, pl.ds(k * LANE, LANE)]
            src_v[t, pl.ds(k * LANE, LANE)] = 2 * v + c
        return carry

    lax.fori_loop(0, n_rows, step, 0)


def _make_sc(T2, l1):
    mesh = plsc.VectorSubcoreMesh(core_axis_name="c", subcore_axis_name="s")
    n_idx_rows = 2 * T2 if l1 else T2

    def body(p_hbm, sh_hbm, src_hbm, dst_hbm, out_hbm,
             src_v, dst_v, bufs, acc_sh, gsems):
        c = lax.axis_index("c")
        s = lax.axis_index("s")
        r0 = s * RPT
        if l1:
            pltpu.sync_copy(src_hbm.at[2 * s], src_v.at[pl.ds(0, T2)])
            pltpu.sync_copy(src_hbm.at[2 * s + 1], src_v.at[pl.ds(T2, T2)])
            pltpu.sync_copy(dst_hbm.at[2 * s], dst_v.at[pl.ds(0, T2)])
            pltpu.sync_copy(dst_hbm.at[2 * s + 1], dst_v.at[pl.ds(T2, T2)])
        else:
            w = c * NS + s
            pltpu.sync_copy(src_hbm.at[w], src_v)
            pltpu.sync_copy(dst_hbm.at[w], dst_v)
        pltpu.sync_copy(sh_hbm.at[pl.ds(r0, RPT), pl.ds(c * 64, 64)],
                        acc_sh.at[pl.ds(r0, RPT)])
        _transform_idx(src_v, c, n_idx_rows)
        plsc.subcore_barrier()
        _edge_loop(p_hbm, src_v, dst_v, acc_sh, bufs, gsems, n_idx_rows)
        plsc.subcore_barrier()
        pltpu.sync_copy(acc_sh.at[pl.ds(r0, RPT)],
                        out_hbm.at[pl.ds(r0, RPT), pl.ds(c * 64, 64)])

    idx_shape = (n_idx_rows, BATCH)
    return pl.kernel(
        body,
        out_type=jax.ShapeDtypeStruct((N_PAD, 128), jnp.float32),
        mesh=mesh,
        compiler_params=pltpu.CompilerParams(use_tc_tiling_on_sc=False),
        scratch_types=[
            pltpu.VMEM(idx_shape, jnp.int32),
            pltpu.VMEM(idx_shape, jnp.int32),
            [pltpu.VMEM((BATCH, 64), jnp.float32) for _ in range(NBUF)],
            pltpu.VMEM_SHARED((N_PAD, 64), jnp.float32),
            [pltpu.SemaphoreType.DMA for _ in range(NBUF)],
        ],
    )


def kernel(x, edge_index, W1_l, b1, W1_r, W2_l, b2, W2_r):
    src = edge_index[0]
    dst = edge_index[1]
    E = src.shape[0]

    T2 = -(-E // (NW * BATCH))
    T2 += T2 % 2
    pad2 = T2 * NW * BATCH - E
    src2 = jnp.concatenate([src, jnp.zeros((pad2,), jnp.int32)]).reshape(NW, T2, BATCH)
    dst2 = jnp.concatenate([dst, jnp.full((pad2,), DUMP, jnp.int32)]).reshape(NW, T2, BATCH)

    p1, s1 = pl.pallas_call(
        _proj_body,
        grid=(25,),
        in_specs=[
            pl.BlockSpec((400, D), lambda i: (i, 0)),
            pl.BlockSpec((D, H), lambda i: (0, 0)),
            pl.BlockSpec((D, H), lambda i: (0, 0)),
            pl.BlockSpec((1, H), lambda i: (0, 0)),
        ],
        out_specs=[pl.BlockSpec((400, H), lambda i: (i, 0)) for _ in range(2)],
        out_shape=[jax.ShapeDtypeStruct((N_PAD, H), jnp.float32) for _ in range(2)],
    )(x, W1_l, W1_r, b1.reshape(1, H))

    acc1 = _make_sc(T2, l1=True)(p1.reshape(2 * N_PAD, 64), s1, src2, dst2)

    p2, s2 = pl.pallas_call(
        _mid_body,
        grid=(8,),
        in_specs=[
            pl.BlockSpec((1280, H), lambda i: (i, 0)),
            pl.BlockSpec((H, C), lambda i: (0, 0)),
            pl.BlockSpec((H, C), lambda i: (0, 0)),
            pl.BlockSpec((1, C), lambda i: (0, 0)),
        ],
        out_specs=[pl.BlockSpec((1280, 128), lambda i: (i, 0)) for _ in range(2)],
        out_shape=[jax.ShapeDtypeStruct((N_PAD, 128), jnp.float32) for _ in range(2)],
    )(acc1, W2_l, W2_r, b2.reshape(1, C))

    acc2 = _make_sc(T2, l1=False)(p2.reshape(2 * N_PAD, 64), s2, src2, dst2)

    out = pl.pallas_call(
        _final_body,
        grid=(5,),
        in_specs=[pl.BlockSpec((2000, 128), lambda i: (i, 0))],
        out_specs=pl.BlockSpec((2000, C), lambda i: (i, 0)),
        out_shape=jax.ShapeDtypeStruct((N, C), jnp.float32),
    )(acc2)
    return out


# speedup vs baseline: 1.4389x; 1.4389x over previous
import jax
import jax.numpy as jnp
from jax import lax
from jax.experimental import pallas as pl
from jax.experimental.pallas import tpu as pltpu
from jax.experimental.pallas import tpu_sc as plsc

N = 10000
D = 128
H = 128
C = 40

NC = 2
NS = 16
NW = NC * NS
BATCH = 128
N_PAD = 10240
DUMP = N
RPT = N_PAD // NS

_HIGH = lax.Precision.HIGHEST


def _proj_body(x_ref, wl_ref, wr_ref, b_ref, p_ref, s_ref):
    xb = x_ref[...]
    p = jnp.dot(xb, wl_ref[...], precision=_HIGH)
    s = jnp.dot(xb, wr_ref[...], precision=_HIGH) + b_ref[...]
    p_ref[0] = p[:, :64]
    p_ref[1] = p[:, 64:]
    s_ref[0] = s[:, :64]
    s_ref[1] = s[:, 64:]


def _mid_body(acc_ref, wl_ref, wr_ref, b_ref, p_ref, s_ref):
    h = jnp.maximum(jnp.concatenate([acc_ref[0], acc_ref[1]], axis=1), 0.0)
    p_ref[...] = jnp.dot(h, wl_ref[...], precision=_HIGH)
    s_ref[...] = 0.5 * (jnp.dot(h, wr_ref[...], precision=_HIGH) + b_ref[...])


def _final_body(acc_ref, o_ref):
    o_ref[...] = acc_ref[0] + acc_ref[1]


NBUF = 2


def _edge_loop(p_hbm, src_v, dst_v, acc_sh, bufs, gsems, ssems, T):
    def step(i, carry):
        j0 = 2 * i
        j1 = j0 + 1
        d0 = pltpu.async_copy(p_hbm.at[src_v.at[j0]], bufs[0], gsems[0])
        d1 = pltpu.async_copy(p_hbm.at[src_v.at[j1]], bufs[1], gsems[1])
        d0.wait()
        pltpu.sync_copy(bufs[0], acc_sh.at[dst_v.at[j0]], add=True)
        d1.wait()
        pltpu.sync_copy(bufs[1], acc_sh.at[dst_v.at[j1]], add=True)
        return carry

    lax.fori_loop(0, T // 2, step, 0)


def _make_sc_l1(T):
    mesh = plsc.VectorSubcoreMesh(core_axis_name="c", subcore_axis_name="s")

    def body(p_hbm, sh_hbm, src0_hbm, src1_hbm, dst_hbm, out_hbm,
             src_v, dst_v, bufs, acc_sh, gsems, ssems):
        c = lax.axis_index("c")
        s = lax.axis_index("s")
        r0 = s * RPT
        pltpu.sync_copy(sh_hbm.at[c, pl.ds(r0, RPT)], acc_sh.at[pl.ds(r0, RPT)])

        @pl.when(c == 0)
        def _():
            pltpu.sync_copy(src0_hbm.at[s], src_v)

        @pl.when(c == 1)
        def _():
            pltpu.sync_copy(src1_hbm.at[s], src_v)

        pltpu.sync_copy(dst_hbm.at[s], dst_v)
        plsc.subcore_barrier()
        _edge_loop(p_hbm, src_v, dst_v, acc_sh, bufs, gsems, ssems, T)
        plsc.subcore_barrier()
        pltpu.sync_copy(acc_sh.at[pl.ds(r0, RPT)], out_hbm.at[c, pl.ds(r0, RPT)])

    return pl.kernel(
        body,
        out_type=jax.ShapeDtypeStruct((NC, N_PAD, 64), jnp.float32),
        mesh=mesh,
        compiler_params=pltpu.CompilerParams(use_tc_tiling_on_sc=False),
        scratch_types=[
            pltpu.VMEM((T, BATCH), jnp.int32),
            pltpu.VMEM((T, BATCH), jnp.int32),
            [pltpu.VMEM((BATCH, 64), jnp.float32) for _ in range(NBUF)],
            pltpu.VMEM_SHARED((N_PAD, 64), jnp.float32),
            [pltpu.SemaphoreType.DMA for _ in range(NBUF)],
            [pltpu.SemaphoreType.DMA for _ in range(NBUF)],
        ],
    )


def _make_sc_l2(T):
    mesh = plsc.VectorSubcoreMesh(core_axis_name="c", subcore_axis_name="s")

    def body(p_hbm, sh_hbm, src_hbm, dst_hbm, out_hbm,
             src_v, dst_v, bufs, acc_sh, gsems, ssems):
        c = lax.axis_index("c")
        s = lax.axis_index("s")
        w = c * NS + s
        r0 = s * RPT
        pltpu.sync_copy(sh_hbm.at[pl.ds(r0, RPT)], acc_sh.at[pl.ds(r0, RPT)])
        pltpu.sync_copy(src_hbm.at[w], src_v)
        pltpu.sync_copy(dst_hbm.at[w], dst_v)
        plsc.subcore_barrier()
        _edge_loop(p_hbm, src_v, dst_v, acc_sh, bufs, gsems, ssems, T)
        plsc.subcore_barrier()
        pltpu.sync_copy(acc_sh.at[pl.ds(r0, RPT)], out_hbm.at[c, pl.ds(r0, RPT)])

    return pl.kernel(
        body,
        out_type=jax.ShapeDtypeStruct((NC, N_PAD, C), jnp.float32),
        mesh=mesh,
        compiler_params=pltpu.CompilerParams(use_tc_tiling_on_sc=False),
        scratch_types=[
            pltpu.VMEM((T, BATCH), jnp.int32),
            pltpu.VMEM((T, BATCH), jnp.int32),
            [pltpu.VMEM((BATCH, C), jnp.float32) for _ in range(NBUF)],
            pltpu.VMEM_SHARED((N_PAD, C), jnp.float32),
            [pltpu.SemaphoreType.DMA for _ in range(NBUF)],
            [pltpu.SemaphoreType.DMA for _ in range(NBUF)],
        ],
    )


def _pad_up(T, m):
    return -(-T // m) * m


def kernel(x, edge_index, W1_l, b1, W1_r, W2_l, b2, W2_r):
    src = edge_index[0]
    dst = edge_index[1]
    E = src.shape[0]

    T1 = _pad_up(-(-E // (NS * BATCH)), NBUF)
    pad1 = T1 * NS * BATCH - E
    src1p = jnp.concatenate([src, jnp.zeros((pad1,), jnp.int32)])
    dst1p = jnp.concatenate([dst, jnp.full((pad1,), DUMP, jnp.int32)])
    src1_a = src1p.reshape(NS, T1, BATCH)
    src1_b = (src1p + N_PAD).reshape(NS, T1, BATCH)
    dst1 = dst1p.reshape(NS, T1, BATCH)

    T2 = _pad_up(-(-E // (NW * BATCH)), NBUF)
    pad2 = T2 * NW * BATCH - E
    src2 = jnp.concatenate([src, jnp.zeros((pad2,), jnp.int32)]).reshape(NW, T2, BATCH)
    dst2 = jnp.concatenate([dst, jnp.full((pad2,), DUMP, jnp.int32)]).reshape(NW, T2, BATCH)

    xp = jnp.pad(x, ((0, N_PAD - N), (0, 0)))
    BR = N_PAD // 16

    p1, s1 = pl.pallas_call(
        _proj_body,
        grid=(16,),
        in_specs=[
            pl.BlockSpec((BR, D), lambda i: (i, 0)),
            pl.BlockSpec((D, H), lambda i: (0, 0)),
            pl.BlockSpec((D, H), lambda i: (0, 0)),
            pl.BlockSpec((1, H), lambda i: (0, 0)),
        ],
        out_specs=[pl.BlockSpec((NC, BR, 64), lambda i: (0, i, 0)),
                   pl.BlockSpec((NC, BR, 64), lambda i: (0, i, 0))],
        out_shape=[jax.ShapeDtypeStruct((NC, N_PAD, 64), jnp.float32),
                   jax.ShapeDtypeStruct((NC, N_PAD, 64), jnp.float32)],
    )(xp, W1_l, W1_r, b1.reshape(1, H))

    acc1 = _make_sc_l1(T1)(p1.reshape(NC * N_PAD, 64), s1, src1_a, src1_b, dst1)

    p2, s2h = pl.pallas_call(
        _mid_body,
        grid=(16,),
        in_specs=[
            pl.BlockSpec((NC, BR, 64), lambda i: (0, i, 0)),
            pl.BlockSpec((H, C), lambda i: (0, 0)),
            pl.BlockSpec((H, C), lambda i: (0, 0)),
            pl.BlockSpec((1, C), lambda i: (0, 0)),
        ],
        out_specs=[pl.BlockSpec((BR, C), lambda i: (i, 0)),
                   pl.BlockSpec((BR, C), lambda i: (i, 0))],
        out_shape=[jax.ShapeDtypeStruct((N_PAD, C), jnp.float32),
                   jax.ShapeDtypeStruct((N_PAD, C), jnp.float32)],
    )(acc1, W2_l, W2_r, b2.reshape(1, C))

    acc2 = _make_sc_l2(T2)(p2, s2h, src2, dst2)

    out = pl.pallas_call(
        _final_body,
        grid=(25,),
        in_specs=[pl.BlockSpec((NC, 400, C), lambda i: (0, i, 0))],
        out_specs=pl.BlockSpec((400, C), lambda i: (i, 0)),
        out_shape=jax.ShapeDtypeStruct((N, C), jnp.float32),
    )(acc2)
    return out


# speedup vs baseline: 1.4612x; 1.0155x over previous
import jax
import jax.numpy as jnp
from jax import lax
from jax.experimental import pallas as pl
from jax.experimental.pallas import tpu as pltpu
from jax.experimental.pallas import tpu_sc as plsc

N = 10000
D = 128
H = 128
C = 40

NC = 2
NS = 16
NW = NC * NS
BATCH = 128
N_PAD = 10240
DUMP = N
RPT = N_PAD // NS

_HIGH = lax.Precision.HIGHEST


def _proj_body(x_ref, wl_ref, wr_ref, b_ref, p_ref, s_ref):
    xb = x_ref[...]
    p = jnp.dot(xb, wl_ref[...], precision=_HIGH)
    s = jnp.dot(xb, wr_ref[...], precision=_HIGH) + b_ref[...]
    p_ref[0] = p[:, :64]
    p_ref[1] = p[:, 64:]
    s_ref[0] = s[:, :64]
    s_ref[1] = s[:, 64:]


def _mid_body(acc_ref, wl_ref, wr_ref, b_ref, p_ref, s_ref):
    h = jnp.maximum(jnp.concatenate([acc_ref[0], acc_ref[1]], axis=1), 0.0)
    p = jnp.dot(h, wl_ref[...], precision=_HIGH)
    p_ref[0] = p
    p_ref[1] = p
    s_ref[...] = 0.5 * (jnp.dot(h, wr_ref[...], precision=_HIGH) + b_ref[...])


def _final_body(acc_ref, o_ref):
    o_ref[...] = acc_ref[0] + acc_ref[1]


NBUF = 2


def _edge_loop(p_hbm, src_v, dst_v, acc_sh, bufs, gsems, ssems, T):
    def step(i, carry):
        j0 = 2 * i
        j1 = j0 + 1
        d0 = pltpu.async_copy(p_hbm.at[src_v.at[j0]], bufs[0], gsems[0])
        d1 = pltpu.async_copy(p_hbm.at[src_v.at[j1]], bufs[1], gsems[1])
        d0.wait()
        pltpu.sync_copy(bufs[0], acc_sh.at[dst_v.at[j0]], add=True)
        d1.wait()
        pltpu.sync_copy(bufs[1], acc_sh.at[dst_v.at[j1]], add=True)
        return carry

    lax.fori_loop(0, T // 2, step, 0)


def _make_sc_l1(T):
    mesh = plsc.VectorSubcoreMesh(core_axis_name="c", subcore_axis_name="s")

    def body(p_hbm, sh_hbm, src0_hbm, src1_hbm, dst_hbm, out_hbm,
             src_v, dst_v, bufs, acc_sh, gsems, ssems):
        c = lax.axis_index("c")
        s = lax.axis_index("s")
        r0 = s * RPT
        pltpu.sync_copy(sh_hbm.at[c, pl.ds(r0, RPT)], acc_sh.at[pl.ds(r0, RPT)])

        @pl.when(c == 0)
        def _():
            pltpu.sync_copy(src0_hbm.at[s], src_v)

        @pl.when(c == 1)
        def _():
            pltpu.sync_copy(src1_hbm.at[s], src_v)

        pltpu.sync_copy(dst_hbm.at[s], dst_v)
        plsc.subcore_barrier()
        _edge_loop(p_hbm, src_v, dst_v, acc_sh, bufs, gsems, ssems, T)
        plsc.subcore_barrier()
        pltpu.sync_copy(acc_sh.at[pl.ds(r0, RPT)], out_hbm.at[c, pl.ds(r0, RPT)])

    return pl.kernel(
        body,
        out_type=jax.ShapeDtypeStruct((NC, N_PAD, 64), jnp.float32),
        mesh=mesh,
        compiler_params=pltpu.CompilerParams(use_tc_tiling_on_sc=False),
        scratch_types=[
            pltpu.VMEM((T, BATCH), jnp.int32),
            pltpu.VMEM((T, BATCH), jnp.int32),
            [pltpu.VMEM((BATCH, 64), jnp.float32) for _ in range(NBUF)],
            pltpu.VMEM_SHARED((N_PAD, 64), jnp.float32),
            [pltpu.SemaphoreType.DMA for _ in range(NBUF)],
            [pltpu.SemaphoreType.DMA for _ in range(NBUF)],
        ],
    )


def _make_sc_l2(T):
    mesh = plsc.VectorSubcoreMesh(core_axis_name="c", subcore_axis_name="s")

    def body(p_hbm, sh_hbm, src0_hbm, src1_hbm, dst_hbm, out_hbm,
             src_v, dst_v, bufs, acc_sh, gsems, ssems):
        c = lax.axis_index("c")
        s = lax.axis_index("s")
        w = c * NS + s
        r0 = s * RPT
        pltpu.sync_copy(sh_hbm.at[pl.ds(r0, RPT)], acc_sh.at[pl.ds(r0, RPT)])

        @pl.when(c == 0)
        def _():
            pltpu.sync_copy(src0_hbm.at[w], src_v)

        @pl.when(c == 1)
        def _():
            pltpu.sync_copy(src1_hbm.at[w], src_v)

        pltpu.sync_copy(dst_hbm.at[w], dst_v)
        plsc.subcore_barrier()
        _edge_loop(p_hbm, src_v, dst_v, acc_sh, bufs, gsems, ssems, T)
        plsc.subcore_barrier()
        pltpu.sync_copy(acc_sh.at[pl.ds(r0, RPT)], out_hbm.at[c, pl.ds(r0, RPT)])

    return pl.kernel(
        body,
        out_type=jax.ShapeDtypeStruct((NC, N_PAD, C), jnp.float32),
        mesh=mesh,
        compiler_params=pltpu.CompilerParams(use_tc_tiling_on_sc=False),
        scratch_types=[
            pltpu.VMEM((T, BATCH), jnp.int32),
            pltpu.VMEM((T, BATCH), jnp.int32),
            [pltpu.VMEM((BATCH, C), jnp.float32) for _ in range(NBUF)],
            pltpu.VMEM_SHARED((N_PAD, C), jnp.float32),
            [pltpu.SemaphoreType.DMA for _ in range(NBUF)],
            [pltpu.SemaphoreType.DMA for _ in range(NBUF)],
        ],
    )


def _pad_up(T, m):
    return -(-T // m) * m


def kernel(x, edge_index, W1_l, b1, W1_r, W2_l, b2, W2_r):
    src = edge_index[0]
    dst = edge_index[1]
    E = src.shape[0]

    T1 = _pad_up(-(-E // (NS * BATCH)), NBUF)
    pad1 = T1 * NS * BATCH - E
    src1p = jnp.concatenate([src, jnp.zeros((pad1,), jnp.int32)])
    dst1p = jnp.concatenate([dst, jnp.full((pad1,), DUMP, jnp.int32)])
    src1_a = src1p.reshape(NS, T1, BATCH)
    src1_b = (src1p + N_PAD).reshape(NS, T1, BATCH)
    dst1 = dst1p.reshape(NS, T1, BATCH)

    T2 = _pad_up(-(-E // (NW * BATCH)), NBUF)
    pad2 = T2 * NW * BATCH - E
    src2p = jnp.concatenate([src, jnp.zeros((pad2,), jnp.int32)])
    src2 = src2p.reshape(NW, T2, BATCH)
    src2b = (src2p + N_PAD).reshape(NW, T2, BATCH)
    dst2 = jnp.concatenate([dst, jnp.full((pad2,), DUMP, jnp.int32)]).reshape(NW, T2, BATCH)

    xp = jnp.pad(x, ((0, N_PAD - N), (0, 0)))
    BR = N_PAD // 16

    p1, s1 = pl.pallas_call(
        _proj_body,
        grid=(16,),
        in_specs=[
            pl.BlockSpec((BR, D), lambda i: (i, 0)),
            pl.BlockSpec((D, H), lambda i: (0, 0)),
            pl.BlockSpec((D, H), lambda i: (0, 0)),
            pl.BlockSpec((1, H), lambda i: (0, 0)),
        ],
        out_specs=[pl.BlockSpec((NC, BR, 64), lambda i: (0, i, 0)),
                   pl.BlockSpec((NC, BR, 64), lambda i: (0, i, 0))],
        out_shape=[jax.ShapeDtypeStruct((NC, N_PAD, 64), jnp.float32),
                   jax.ShapeDtypeStruct((NC, N_PAD, 64), jnp.float32)],
    )(xp, W1_l, W1_r, b1.reshape(1, H))

    acc1 = _make_sc_l1(T1)(p1.reshape(NC * N_PAD, 64), s1, src1_a, src1_b, dst1)

    p2, s2h = pl.pallas_call(
        _mid_body,
        grid=(16,),
        in_specs=[
            pl.BlockSpec((NC, BR, 64), lambda i: (0, i, 0)),
            pl.BlockSpec((H, C), lambda i: (0, 0)),
            pl.BlockSpec((H, C), lambda i: (0, 0)),
            pl.BlockSpec((1, C), lambda i: (0, 0)),
        ],
        out_specs=[pl.BlockSpec((NC, BR, C), lambda i: (0, i, 0)),
                   pl.BlockSpec((BR, C), lambda i: (i, 0))],
        out_shape=[jax.ShapeDtypeStruct((NC, N_PAD, C), jnp.float32),
                   jax.ShapeDtypeStruct((N_PAD, C), jnp.float32)],
    )(acc1, W2_l, W2_r, b2.reshape(1, C))

    acc2 = _make_sc_l2(T2)(p2.reshape(NC * N_PAD, C), s2h, src2, src2b, dst2)

    out = pl.pallas_call(
        _final_body,
        grid=(25,),
        in_specs=[pl.BlockSpec((NC, 400, C), lambda i: (0, i, 0))],
        out_specs=pl.BlockSpec((400, C), lambda i: (i, 0)),
        out_shape=jax.ShapeDtypeStruct((N, C), jnp.float32),
    )(acc2)
    return out


# speedup vs baseline: 2.1657x; 1.4822x over previous
import jax
import jax.numpy as jnp
from jax import lax
from jax.experimental import pallas as pl
from jax.experimental.pallas import tpu as pltpu
from jax.experimental.pallas import tpu_sc as plsc

N = 10000
D = 128
H = 128
C = 40

NC = 2
NS = 16
NW = NC * NS
BATCH = 128
N_PAD = 10240
DUMP = N
RPT = N_PAD // NS

_HIGH = lax.Precision.HIGHEST


def _proj_body(x_ref, wl_ref, wr_ref, b_ref, p_ref, s_ref):
    xb = x_ref[...]
    p = jnp.dot(xb, wl_ref[...], precision=_HIGH)
    s = jnp.dot(xb, wr_ref[...], precision=_HIGH) + b_ref[...]
    p_ref[0] = p[:, :64]
    p_ref[1] = p[:, 64:]
    s_ref[0] = s[:, :64]
    s_ref[1] = s[:, 64:]


def _mid_body(acc_ref, wl_ref, wr_ref, b_ref, p_ref, s_ref):
    h = jnp.maximum(jnp.concatenate([acc_ref[0], acc_ref[1]], axis=1), 0.0)
    p = jnp.dot(h, wl_ref[...], precision=_HIGH)
    p_ref[0] = p
    p_ref[1] = p
    s_ref[...] = 0.5 * (jnp.dot(h, wr_ref[...], precision=_HIGH) + b_ref[...])


def _final_body(acc_ref, o_ref):
    o_ref[...] = acc_ref[0] + acc_ref[1]


NBUF = 2


def _edge_loop(p_hbm, src_v, dst_v, acc_sh, bufs, gsems, ssems, T):
    def step(i, carry):
        j0 = 2 * i
        j1 = j0 + 1
        d0 = pltpu.async_copy(p_hbm.at[src_v.at[j0]], bufs[0], gsems[0])
        d1 = pltpu.async_copy(p_hbm.at[src_v.at[j1]], bufs[1], gsems[1])
        d0.wait()
        pltpu.sync_copy(bufs[0], acc_sh.at[dst_v.at[j0]], add=True)
        d1.wait()
        pltpu.sync_copy(bufs[1], acc_sh.at[dst_v.at[j1]], add=True)
        return carry

    lax.fori_loop(0, T // 2, step, 0)


def _make_sc_l1(T, extra):
    mesh = plsc.VectorSubcoreMesh(core_axis_name="c", subcore_axis_name="s")

    def body(p_hbm, sh_hbm, src0_hbm, src1_hbm, dst_hbm, out_hbm,
             src_v, dst_v, bufs, acc_sh, gsems, ssems):
        c = lax.axis_index("c")
        s = lax.axis_index("s")
        r0 = s * RPT
        pltpu.sync_copy(sh_hbm.at[c, pl.ds(r0, RPT)], acc_sh.at[pl.ds(r0, RPT)])

        @pl.when(c == 0)
        def _():
            pltpu.sync_copy(src0_hbm.at[pl.ds(s * T, T)], src_v.at[pl.ds(0, T)])

        @pl.when(c == 1)
        def _():
            pltpu.sync_copy(src1_hbm.at[pl.ds(s * T, T)], src_v.at[pl.ds(0, T)])

        pltpu.sync_copy(dst_hbm.at[pl.ds(s * T, T)], dst_v.at[pl.ds(0, T)])

        @pl.when(s < extra)
        def _():
            @pl.when(c == 0)
            def _():
                pltpu.sync_copy(src0_hbm.at[pl.ds(NS * T + s, 1)],
                                src_v.at[pl.ds(T, 1)])

            @pl.when(c == 1)
            def _():
                pltpu.sync_copy(src1_hbm.at[pl.ds(NS * T + s, 1)],
                                src_v.at[pl.ds(T, 1)])

            pltpu.sync_copy(dst_hbm.at[pl.ds(NS * T + s, 1)],
                            dst_v.at[pl.ds(T, 1)])

        plsc.subcore_barrier()
        _edge_loop(p_hbm, src_v, dst_v, acc_sh, bufs, gsems, ssems, T)

        @pl.when(s < extra)
        def _():
            d = pltpu.async_copy(p_hbm.at[src_v.at[T]], bufs[0], gsems[0])
            d.wait()
            pltpu.sync_copy(bufs[0], acc_sh.at[dst_v.at[T]], add=True)

        plsc.subcore_barrier()
        pltpu.sync_copy(acc_sh.at[pl.ds(r0, RPT)], out_hbm.at[c, pl.ds(r0, RPT)])

    return pl.kernel(
        body,
        out_type=jax.ShapeDtypeStruct((NC, N_PAD, 64), jnp.float32),
        mesh=mesh,
        compiler_params=pltpu.CompilerParams(use_tc_tiling_on_sc=False),
        scratch_types=[
            pltpu.VMEM((T + 1, BATCH), jnp.int32),
            pltpu.VMEM((T + 1, BATCH), jnp.int32),
            [pltpu.VMEM((BATCH, 64), jnp.float32) for _ in range(NBUF)],
            pltpu.VMEM_SHARED((N_PAD, 64), jnp.float32),
            [pltpu.SemaphoreType.DMA for _ in range(NBUF)],
            [pltpu.SemaphoreType.DMA for _ in range(NBUF)],
        ],
    )


def _make_sc_l2(T, extra):
    mesh = plsc.VectorSubcoreMesh(core_axis_name="c", subcore_axis_name="s")

    def body(p_hbm, sh_hbm, src0_hbm, src1_hbm, dst_hbm, out_hbm,
             src_v, dst_v, bufs, acc_sh, gsems, ssems):
        c = lax.axis_index("c")
        s = lax.axis_index("s")
        w = c * NS + s
        r0 = s * RPT
        pltpu.sync_copy(sh_hbm.at[pl.ds(r0, RPT)], acc_sh.at[pl.ds(r0, RPT)])

        @pl.when(c == 0)
        def _():
            pltpu.sync_copy(src0_hbm.at[pl.ds(w * T, T)], src_v.at[pl.ds(0, T)])

        @pl.when(c == 1)
        def _():
            pltpu.sync_copy(src1_hbm.at[pl.ds(w * T, T)], src_v.at[pl.ds(0, T)])

        pltpu.sync_copy(dst_hbm.at[pl.ds(w * T, T)], dst_v.at[pl.ds(0, T)])

        @pl.when(w < extra)
        def _():
            @pl.when(c == 0)
            def _():
                pltpu.sync_copy(src0_hbm.at[pl.ds(NW * T + w, 1)],
                                src_v.at[pl.ds(T, 1)])

            @pl.when(c == 1)
            def _():
                pltpu.sync_copy(src1_hbm.at[pl.ds(NW * T + w, 1)],
                                src_v.at[pl.ds(T, 1)])

            pltpu.sync_copy(dst_hbm.at[pl.ds(NW * T + w, 1)],
                            dst_v.at[pl.ds(T, 1)])

        plsc.subcore_barrier()
        _edge_loop(p_hbm, src_v, dst_v, acc_sh, bufs, gsems, ssems, T)

        @pl.when(w < extra)
        def _():
            d = pltpu.async_copy(p_hbm.at[src_v.at[T]], bufs[0], gsems[0])
            d.wait()
            pltpu.sync_copy(bufs[0], acc_sh.at[dst_v.at[T]], add=True)

        plsc.subcore_barrier()
        pltpu.sync_copy(acc_sh.at[pl.ds(r0, RPT)], out_hbm.at[c, pl.ds(r0, RPT)])

    return pl.kernel(
        body,
        out_type=jax.ShapeDtypeStruct((NC, N_PAD, C), jnp.float32),
        mesh=mesh,
        compiler_params=pltpu.CompilerParams(use_tc_tiling_on_sc=False),
        scratch_types=[
            pltpu.VMEM((T + 1, BATCH), jnp.int32),
            pltpu.VMEM((T + 1, BATCH), jnp.int32),
            [pltpu.VMEM((BATCH, C), jnp.float32) for _ in range(NBUF)],
            pltpu.VMEM_SHARED((N_PAD, C), jnp.float32),
            [pltpu.SemaphoreType.DMA for _ in range(NBUF)],
            [pltpu.SemaphoreType.DMA for _ in range(NBUF)],
        ],
    )


def _pad_up(T, m):
    return -(-T // m) * m


def kernel(x, edge_index, W1_l, b1, W1_r, W2_l, b2, W2_r):
    src = edge_index[0]
    dst = edge_index[1]
    E = src.shape[0]

    rows = E // BATCH
    src_r = src.reshape(rows, BATCH)
    srcb_r = src_r + N_PAD
    dst_r = dst.reshape(rows, BATCH)
    T1, X1 = rows // NS, rows % NS
    T2, X2 = rows // NW, rows % NW

    xp = jnp.pad(x, ((0, N_PAD - N), (0, 0)))
    BR = N_PAD // 16

    p1, s1 = pl.pallas_call(
        _proj_body,
        grid=(16,),
        in_specs=[
            pl.BlockSpec((BR, D), lambda i: (i, 0)),
            pl.BlockSpec((D, H), lambda i: (0, 0)),
            pl.BlockSpec((D, H), lambda i: (0, 0)),
            pl.BlockSpec((1, H), lambda i: (0, 0)),
        ],
        out_specs=[pl.BlockSpec((NC, BR, 64), lambda i: (0, i, 0)),
                   pl.BlockSpec((NC, BR, 64), lambda i: (0, i, 0))],
        out_shape=[jax.ShapeDtypeStruct((NC, N_PAD, 64), jnp.float32),
                   jax.ShapeDtypeStruct((NC, N_PAD, 64), jnp.float32)],
    )(xp, W1_l, W1_r, b1.reshape(1, H))

    acc1 = _make_sc_l1(T1, X1)(p1.reshape(NC * N_PAD, 64), s1, src_r, srcb_r, dst_r)

    p2, s2h = pl.pallas_call(
        _mid_body,
        grid=(16,),
        in_specs=[
            pl.BlockSpec((NC, BR, 64), lambda i: (0, i, 0)),
            pl.BlockSpec((H, C), lambda i: (0, 0)),
            pl.BlockSpec((H, C), lambda i: (0, 0)),
            pl.BlockSpec((1, C), lambda i: (0, 0)),
        ],
        out_specs=[pl.BlockSpec((NC, BR, C), lambda i: (0, i, 0)),
                   pl.BlockSpec((BR, C), lambda i: (i, 0))],
        out_shape=[jax.ShapeDtypeStruct((NC, N_PAD, C), jnp.float32),
                   jax.ShapeDtypeStruct((N_PAD, C), jnp.float32)],
    )(acc1, W2_l, W2_r, b2.reshape(1, C))

    acc2 = _make_sc_l2(T2, X2)(p2.reshape(NC * N_PAD, C), s2h, src_r, srcb_r, dst_r)

    out = pl.pallas_call(
        _final_body,
        grid=(25,),
        in_specs=[pl.BlockSpec((NC, 400, C), lambda i: (0, i, 0))],
        out_specs=pl.BlockSpec((400, C), lambda i: (i, 0)),
        out_shape=jax.ShapeDtypeStruct((N, C), jnp.float32),
    )(acc2)
    return out
